# baseline (speedup 1.0000x reference)
import jax
import jax.numpy as jnp
from jax.experimental import pallas as pl
from jax.experimental.pallas import tpu as pltpu


def _matmul_kernel(x_ref, m_ref, o_ref):
    o_ref[...] = jnp.dot(x_ref[...], m_ref[...],
                         preferred_element_type=jnp.float32)


def kernel(x, M):
    B, C, H, W = x.shape
    K = H * W
    N = M.shape[1]
    Ho = 2 * H
    Wo = N // Ho

    xf = x.reshape(B * C, K).astype(jnp.bfloat16)
    Mb = M.astype(jnp.bfloat16)

    BM = 256
    yf = pl.pallas_call(
        _matmul_kernel,
        out_shape=jax.ShapeDtypeStruct((B * C, N), jnp.float32),
        grid=(B * C // BM,),
        in_specs=[
            pl.BlockSpec((BM, K), lambda i: (i, 0)),
            pl.BlockSpec((K, N), lambda i: (0, 0)),
        ],
        out_specs=pl.BlockSpec((BM, N), lambda i: (i, 0)),
        compiler_params=pltpu.CompilerParams(
            dimension_semantics=("parallel",)),
    )(xf, Mb)
    return yf.reshape(B, C, Ho, Wo)
